# baseline (speedup 1.0000x reference)
import functools

import jax
import jax.numpy as jnp
from jax import lax
from jax.experimental import pallas as pl
from jax.experimental.pallas import tpu as pltpu
from jax.experimental.pallas import tpu_sc as plsc

_B = 16384
_D = 32
_NC = 2
_NS = 16
_NW = _NC * _NS
_BPW = _B // _NW
_CH = 128
_NCH = _BPW // _CH
_IDS_ROWS = _B // _CH

_BLK = 2048


def _gather_body(uid_hbm, mid_hbm, ut_hbm, mt_hbm, outu_hbm, outm_hbm,
                 uidx_v, midx_v, urows_v, mrows_v, usem, msem):
    wid = lax.axis_index("s") * _NC + lax.axis_index("c")
    base = wid * _BPW
    crow = wid * _NCH
    pltpu.sync_copy(uid_hbm.at[pl.ds(crow, _NCH)], uidx_v)
    pltpu.sync_copy(mid_hbm.at[pl.ds(crow, _NCH)], midx_v)
    ucps = [pltpu.async_copy(ut_hbm.at[uidx_v.at[j]],
                             urows_v.at[pl.ds(j * _CH, _CH)], usem)
            for j in range(_NCH)]
    mcps = [pltpu.async_copy(mt_hbm.at[midx_v.at[j]],
                             mrows_v.at[pl.ds(j * _CH, _CH)], msem)
            for j in range(_NCH)]
    for c in ucps:
        c.wait()
    for c in mcps:
        c.wait()
    pltpu.sync_copy(urows_v, outu_hbm.at[pl.ds(base, _BPW)])
    pltpu.sync_copy(mrows_v, outm_hbm.at[pl.ds(base, _BPW)])


@functools.cache
def _gather():
    return pl.kernel(
        _gather_body,
        out_type=[jax.ShapeDtypeStruct((_B, _D), jnp.float32),
                  jax.ShapeDtypeStruct((_B, _D), jnp.float32)],
        mesh=plsc.VectorSubcoreMesh(core_axis_name="c", subcore_axis_name="s"),
        scratch_types=[pltpu.VMEM((_NCH, _CH), jnp.int32),
                       pltpu.VMEM((_NCH, _CH), jnp.int32),
                       pltpu.VMEM((_BPW, _D), jnp.float32),
                       pltpu.VMEM((_BPW, _D), jnp.float32),
                       pltpu.SemaphoreType.DMA,
                       pltpu.SemaphoreType.DMA],
        compiler_params=pltpu.CompilerParams(use_tc_tiling_on_sc=False),
    )


def _mlp_body(u_ref, m_ref, w1u_ref, w1m_ref, b1_ref, w2_ref, b2_ref,
              w3_ref, b3_ref, wo_ref, bo_ref, o_ref):
    x = (jnp.dot(u_ref[...], w1u_ref[...], preferred_element_type=jnp.float32)
         + jnp.dot(m_ref[...], w1m_ref[...], preferred_element_type=jnp.float32)
         + b1_ref[...])
    x = jnp.maximum(x, 0.0)
    x = jnp.maximum(
        jnp.dot(x, w2_ref[...], preferred_element_type=jnp.float32) + b2_ref[...], 0.0)
    x = jnp.maximum(
        jnp.dot(x, w3_ref[...], preferred_element_type=jnp.float32) + b3_ref[...], 0.0)
    z = jnp.dot(x, wo_ref[...], preferred_element_type=jnp.float32) + bo_ref[...]
    o_ref[...] = 1.0 / (1.0 + jnp.exp(-z))


def _mlp(u, m, w1u, w1m, b1, w2, b2, w3, b3, wo, bo):
    grid = (_B // _BLK,)
    full = lambda shape: pl.BlockSpec(shape, lambda i: (0, 0))
    return pl.pallas_call(
        _mlp_body,
        grid=grid,
        in_specs=[
            pl.BlockSpec((_BLK, _D), lambda i: (i, 0)),
            pl.BlockSpec((_BLK, _D), lambda i: (i, 0)),
            full(w1u.shape), full(w1m.shape), full(b1.shape),
            full(w2.shape), full(b2.shape),
            full(w3.shape), full(b3.shape),
            full(wo.shape), full(bo.shape),
        ],
        out_specs=pl.BlockSpec((_BLK, 1), lambda i: (i, 0)),
        out_shape=jax.ShapeDtypeStruct((_B, 1), jnp.float32),
    )(u, m, w1u, w1m, b1, w2, b2, w3, b3, wo, bo)


def kernel(user_ids, movie_ids, user_table, movie_table,
           W1, b1, W2, b2, W3, b3, Wo, bo):
    uid = user_ids.astype(jnp.int32).reshape(_IDS_ROWS, _CH)
    mid = movie_ids.astype(jnp.int32).reshape(_IDS_ROWS, _CH)
    u, m = _gather()(uid, mid, user_table, movie_table)
    out = _mlp(u, m,
               W1[:_D, :], W1[_D:, :], b1.reshape(1, -1),
               W2, b2.reshape(1, -1),
               W3, b3.reshape(1, -1),
               Wo, bo.reshape(1, -1))
    return out[:, 0]
